# baseline (speedup 1.0000x reference)
import functools

import jax
import jax.numpy as jnp
from jax import lax
from jax.experimental import pallas as pl
from jax.experimental.pallas import tpu as pltpu
from jax.experimental.pallas import tpu_sc as plsc

NUM_TOKENS = 8192
DIM = 64
BATCH = 8
SEQ = 1024
TOTAL = BATCH * SEQ

_info = plsc.get_sparse_core_info()
_NC, _NS = 1, _info.num_subcores
_NW = _NC * _NS
_PER_W = TOTAL // _NW
_NCHUNK = 2
_CHUNK = _PER_W // _NCHUNK


@functools.partial(
    pl.kernel,
    mesh=plsc.VectorSubcoreMesh(
        core_axis_name="c", subcore_axis_name="s", num_cores=1
    ),
    out_type=jax.ShapeDtypeStruct((BATCH, SEQ, DIM), jnp.float32),
    scratch_types=[
        pltpu.VMEM((_PER_W,), jnp.int32),
        pltpu.VMEM((_PER_W, DIM), jnp.float32),
        pltpu.SemaphoreType.DMA,
        pltpu.SemaphoreType.DMA,
        pltpu.SemaphoreType.DMA,
    ],
    compiler_params=pltpu.CompilerParams(
        use_tc_tiling_on_sc=False,
        disable_bounds_checks=True,
        disable_semaphore_checks=True,
    ),
)
def _gather_kernel(idx_hbm, table_hbm, out_hbm, idx_v, rows_v, sem_i, sem_g, sem_w):
    wid = lax.axis_index("s")
    b = wid // (SEQ // _PER_W)
    to = (wid % (SEQ // _PER_W)) * _PER_W
    idx_copies = []
    for c in range(_NCHUNK):
        lo = c * _CHUNK
        idx_copies.append(
            pltpu.async_copy(
                idx_hbm.at[b, pl.ds(to + lo, _CHUNK)],
                idx_v.at[pl.ds(lo, _CHUNK)],
                sem_i,
            )
        )
    gathers = []
    for c in range(_NCHUNK):
        lo = c * _CHUNK
        idx_copies[c].wait()
        gathers.append(
            pltpu.async_copy(
                table_hbm.at[idx_v.at[pl.ds(lo, _CHUNK)]],
                rows_v.at[pl.ds(lo, _CHUNK)],
                sem_g,
            )
        )
    writes = []
    for c in range(_NCHUNK):
        lo = c * _CHUNK
        gathers[c].wait()
        writes.append(
            pltpu.async_copy(
                rows_v.at[pl.ds(lo, _CHUNK)],
                out_hbm.at[b, pl.ds(to + lo, _CHUNK)],
                sem_w,
            )
        )
    for c in range(_NCHUNK):
        writes[c].wait()


@jax.jit
def kernel(embed_id, weight):
    return _gather_kernel(embed_id.astype(jnp.int32), weight)


# speedup vs baseline: 1.0005x; 1.0005x over previous
import functools

import jax
import jax.numpy as jnp
from jax import lax
from jax.experimental import pallas as pl
from jax.experimental.pallas import tpu as pltpu
from jax.experimental.pallas import tpu_sc as plsc

NUM_TOKENS = 8192
DIM = 64
BATCH = 8
SEQ = 1024
TOTAL = BATCH * SEQ

_info = plsc.get_sparse_core_info()
_NC, _NS = 1, _info.num_subcores
_NW = _NC * _NS
_PER_W = TOTAL // _NW
_NCHUNK = 4
_CHUNK = _PER_W // _NCHUNK


@functools.partial(
    pl.kernel,
    mesh=plsc.VectorSubcoreMesh(
        core_axis_name="c", subcore_axis_name="s", num_cores=1
    ),
    out_type=jax.ShapeDtypeStruct((BATCH, SEQ, DIM), jnp.float32),
    scratch_types=[
        pltpu.VMEM((_PER_W,), jnp.int32),
        pltpu.VMEM((_PER_W, DIM), jnp.float32),
        pltpu.SemaphoreType.DMA,
        pltpu.SemaphoreType.DMA,
        pltpu.SemaphoreType.DMA,
    ],
    compiler_params=pltpu.CompilerParams(
        use_tc_tiling_on_sc=False,
        disable_bounds_checks=True,
        disable_semaphore_checks=True,
    ),
)
def _gather_kernel(idx_hbm, table_hbm, out_hbm, idx_v, rows_v, sem_i, sem_g, sem_w):
    wid = lax.axis_index("s")
    b = wid // (SEQ // _PER_W)
    to = (wid % (SEQ // _PER_W)) * _PER_W
    idx_copies = []
    for c in range(_NCHUNK):
        lo = c * _CHUNK
        idx_copies.append(
            pltpu.async_copy(
                idx_hbm.at[b, pl.ds(to + lo, _CHUNK)],
                idx_v.at[pl.ds(lo, _CHUNK)],
                sem_i,
            )
        )
    gathers = []
    for c in range(_NCHUNK):
        lo = c * _CHUNK
        idx_copies[c].wait()
        gathers.append(
            pltpu.async_copy(
                table_hbm.at[idx_v.at[pl.ds(lo, _CHUNK)]],
                rows_v.at[pl.ds(lo, _CHUNK)],
                sem_g,
            )
        )
    writes = []
    for c in range(_NCHUNK):
        lo = c * _CHUNK
        gathers[c].wait()
        writes.append(
            pltpu.async_copy(
                rows_v.at[pl.ds(lo, _CHUNK)],
                out_hbm.at[b, pl.ds(to + lo, _CHUNK)],
                sem_w,
            )
        )
    for c in range(_NCHUNK):
        writes[c].wait()


@jax.jit
def kernel(embed_id, weight):
    return _gather_kernel(embed_id.astype(jnp.int32), weight)


# speedup vs baseline: 1.0027x; 1.0022x over previous
import functools

import jax
import jax.numpy as jnp
from jax import lax
from jax.experimental import pallas as pl
from jax.experimental.pallas import tpu as pltpu
from jax.experimental.pallas import tpu_sc as plsc

NUM_TOKENS = 8192
DIM = 64
BATCH = 8
SEQ = 1024
TOTAL = BATCH * SEQ

_info = plsc.get_sparse_core_info()
_NC, _NS = 1, _info.num_subcores
_NW = _NC * _NS
_PER_W = TOTAL // _NW
_NCHUNK = 4
_CHUNK = _PER_W // _NCHUNK


@functools.partial(
    pl.kernel,
    mesh=plsc.VectorSubcoreMesh(
        core_axis_name="c", subcore_axis_name="s", num_cores=1
    ),
    out_type=jax.ShapeDtypeStruct((BATCH, SEQ, DIM), jnp.float32),
    scratch_types=[
        pltpu.VMEM((_PER_W,), jnp.int32),
        pltpu.VMEM((_PER_W, DIM), jnp.float32),
        pltpu.SemaphoreType.DMA,
        pltpu.SemaphoreType.DMA,
        pltpu.SemaphoreType.DMA,
    ],
    compiler_params=pltpu.CompilerParams(use_tc_tiling_on_sc=False),
)
def _gather_kernel(idx_hbm, table_hbm, out_hbm, idx_v, rows_v, sem_i, sem_g, sem_w):
    wid = lax.axis_index("s")
    b = wid // (SEQ // _PER_W)
    to = (wid % (SEQ // _PER_W)) * _PER_W
    idx_copies = []
    for c in range(_NCHUNK):
        lo = c * _CHUNK
        idx_copies.append(
            pltpu.async_copy(
                idx_hbm.at[b, pl.ds(to + lo, _CHUNK)],
                idx_v.at[pl.ds(lo, _CHUNK)],
                sem_i,
            )
        )
    gathers = []
    for c in range(_NCHUNK):
        lo = c * _CHUNK
        idx_copies[c].wait()
        gathers.append(
            pltpu.async_copy(
                table_hbm.at[idx_v.at[pl.ds(lo, _CHUNK)]],
                rows_v.at[pl.ds(lo, _CHUNK)],
                sem_g,
            )
        )
    writes = []
    for c in range(_NCHUNK):
        lo = c * _CHUNK
        gathers[c].wait()
        writes.append(
            pltpu.async_copy(
                rows_v.at[pl.ds(lo, _CHUNK)],
                out_hbm.at[b, pl.ds(to + lo, _CHUNK)],
                sem_w,
            )
        )
    for c in range(_NCHUNK):
        writes[c].wait()


@jax.jit
def kernel(embed_id, weight):
    return _gather_kernel(embed_id.astype(jnp.int32), weight)
